# Initial kernel scaffold; baseline (speedup 1.0000x reference)
import functools

import jax
import jax.numpy as jnp
from jax import lax
from jax.experimental import pallas as pl
from jax.experimental.pallas import tpu as pltpu
from jax.experimental.pallas import tpu_sc as plsc

N_LAYERS = 3
N = 10000
E = 320000
D = 128
DH = D // 2
NC, NS = 2, 16
CHUNK = 128
N_CHUNKS = 160
E_TILE = CHUNK * N_CHUNKS
E_PAD = NS * E_TILE
ROWS_TILE = N // NS
ZROWS = 125


def _body(x_hbm, col_hbm, row_hbm, vals_hbm, out_hbm, tmp_hbm,
          col_v, row_v, vals_v, gbuf, zbuf, acc):
    c = lax.axis_index("c")
    s = lax.axis_index("s")

    pltpu.sync_copy(col_hbm.at[s], col_v)
    pltpu.sync_copy(row_hbm.at[s], row_v)
    pltpu.sync_copy(vals_hbm.at[s], vals_v)

    offv = lax.broadcast(c.astype(jnp.int32) * N, (16,))

    def adj_body(r, carry):
        for q in range(CHUNK // 16):
            sl = pl.ds(q * 16, 16)
            col_v[r, sl] = col_v[r, sl] + offv
        return carry

    lax.fori_loop(0, N_CHUNKS, adj_body, 0)

    def zb_body(r, carry):
        for q in range(DH // 16):
            zbuf[r, pl.ds(q * 16, 16)] = jnp.zeros((16,), jnp.float32)
        return carry

    lax.fori_loop(0, ZROWS, zb_body, 0)

    for layer in range(N_LAYERS):
        src = (x_hbm, tmp_hbm, out_hbm)[layer]
        dst = (tmp_hbm, out_hbm, out_hbm)[layer]

        for i in range(ROWS_TILE // ZROWS):
            pltpu.sync_copy(zbuf, acc.at[pl.ds(s * ROWS_TILE + i * ZROWS, ZROWS)])
        plsc.subcore_barrier()

        def chunk_body(k, carry):
            pltpu.sync_copy(src.at[col_v.at[k]], gbuf)

            def edge_body(j, carry2):
                v = vals_v[k, j]
                vb = lax.broadcast(v, (16,))
                for d in range(DH // 16):
                    sl = pl.ds(d * 16, 16)
                    gbuf[j, sl] = gbuf[j, sl] * vb
                return carry2

            lax.fori_loop(0, CHUNK, edge_body, 0)

            pltpu.sync_copy(gbuf, acc.at[row_v.at[k]], add=True)
            return carry

        lax.fori_loop(0, N_CHUNKS, chunk_body, 0)
        plsc.subcore_barrier()

        pltpu.sync_copy(
            acc.at[pl.ds(s * ROWS_TILE, ROWS_TILE)],
            dst.at[pl.ds(c * N + s * ROWS_TILE, ROWS_TILE)],
        )
        plsc.subcore_barrier()


@jax.jit
def kernel(x, edge_row, edge_col, edge_vals):
    x2 = x.reshape(N, NC, DH).transpose(1, 0, 2).reshape(NC * N, DH)
    pad = E_PAD - E
    col3 = jnp.concatenate([edge_col, jnp.zeros((pad,), jnp.int32)]
                           ).reshape(NS, N_CHUNKS, CHUNK)
    row3 = jnp.concatenate([edge_row, jnp.zeros((pad,), jnp.int32)]
                           ).reshape(NS, N_CHUNKS, CHUNK)
    vals3 = jnp.concatenate([edge_vals, jnp.zeros((pad,), jnp.float32)]
                            ).reshape(NS, N_CHUNKS, CHUNK)

    mesh = plsc.VectorSubcoreMesh(core_axis_name="c", subcore_axis_name="s")
    out, _tmp = pl.kernel(
        _body,
        out_type=(
            jax.ShapeDtypeStruct((NC * N, DH), jnp.float32),
            jax.ShapeDtypeStruct((NC * N, DH), jnp.float32),
        ),
        mesh=mesh,
        scratch_types=[
            pltpu.VMEM((N_CHUNKS, CHUNK), jnp.int32),
            pltpu.VMEM((N_CHUNKS, CHUNK), jnp.int32),
            pltpu.VMEM((N_CHUNKS, CHUNK), jnp.float32),
            pltpu.VMEM((CHUNK, DH), jnp.float32),
            pltpu.VMEM((ZROWS, DH), jnp.float32),
            pltpu.VMEM_SHARED((N, DH), jnp.float32),
        ],
    )(x2, col3, row3, vals3)

    return out.reshape(NC, N, DH).transpose(1, 0, 2).reshape(N, D)


# speedup vs baseline: 3.4493x; 3.4493x over previous
import functools

import jax
import jax.numpy as jnp
from jax import lax
from jax.experimental import pallas as pl
from jax.experimental.pallas import tpu as pltpu
from jax.experimental.pallas import tpu_sc as plsc

N_LAYERS = 3
N = 10000
E = 320000
D = 128
DH = D // 2
NC, NS = 2, 16
CHUNK = 128
N_CHUNKS = 160
E_TILE = CHUNK * N_CHUNKS
E_PAD = NS * E_TILE
NP = 10240
ROWS_TILE = NP // NS
ZROWS = 128


def _body(x_hbm, col_hbm, row_hbm, vals_hbm, out_hbm, tmp_hbm,
          col_v, row_v, vals_v, gbuf, zbuf, acc):
    c = lax.axis_index("c")
    s = lax.axis_index("s")

    pltpu.sync_copy(col_hbm.at[s], col_v)
    pltpu.sync_copy(row_hbm.at[s], row_v)
    pltpu.sync_copy(vals_hbm.at[s], vals_v)

    offv = lax.broadcast(c.astype(jnp.int32) * NP, (16,))

    def adj_body(r, carry):
        for q in range(CHUNK // 16):
            sl = pl.ds(q * 16, 16)
            col_v[r, sl] = col_v[r, sl] + offv
        return carry

    lax.fori_loop(0, N_CHUNKS, adj_body, 0)

    def zb_body(r, carry):
        for q in range(DH // 16):
            zbuf[r, pl.ds(q * 16, 16)] = jnp.zeros((16,), jnp.float32)
        return carry

    lax.fori_loop(0, ZROWS, zb_body, 0)

    for layer in range(N_LAYERS):
        src = (x_hbm, tmp_hbm, out_hbm)[layer]
        dst = (tmp_hbm, out_hbm, out_hbm)[layer]

        for i in range(ROWS_TILE // ZROWS):
            pltpu.sync_copy(zbuf, acc.at[pl.ds(s * ROWS_TILE + i * ZROWS, ZROWS)])
        plsc.subcore_barrier()

        def chunk_body(k, carry):
            pltpu.sync_copy(src.at[col_v.at[k]], gbuf)

            def grp_body(g, carry2):
                val16 = vals_v[k, pl.ds(g * 16, 16)]
                for e in range(16):
                    vb = lax.broadcast(val16[e], (16,))
                    j = g * 16 + e
                    for d in range(DH // 16):
                        sl = pl.ds(d * 16, 16)
                        gbuf[j, sl] = gbuf[j, sl] * vb
                return carry2

            lax.fori_loop(0, CHUNK // 16, grp_body, 0)

            pltpu.sync_copy(gbuf, acc.at[row_v.at[k]], add=True)
            return carry

        lax.fori_loop(0, N_CHUNKS, chunk_body, 0)
        plsc.subcore_barrier()

        pltpu.sync_copy(
            acc.at[pl.ds(s * ROWS_TILE, ROWS_TILE)],
            dst.at[pl.ds(c * NP + s * ROWS_TILE, ROWS_TILE)],
        )
        plsc.subcore_barrier()


@jax.jit
def kernel(x, edge_row, edge_col, edge_vals):
    x2 = x.reshape(N, NC, DH).transpose(1, 0, 2)
    x2 = jnp.pad(x2, ((0, 0), (0, NP - N), (0, 0))).reshape(NC * NP, DH)
    pad = E_PAD - E
    col3 = jnp.concatenate([edge_col, jnp.zeros((pad,), jnp.int32)]
                           ).reshape(NS, N_CHUNKS, CHUNK)
    row3 = jnp.concatenate([edge_row, jnp.zeros((pad,), jnp.int32)]
                           ).reshape(NS, N_CHUNKS, CHUNK)
    vals3 = jnp.concatenate([edge_vals, jnp.zeros((pad,), jnp.float32)]
                            ).reshape(NS, N_CHUNKS, CHUNK)

    mesh = plsc.VectorSubcoreMesh(core_axis_name="c", subcore_axis_name="s")
    out, _tmp = pl.kernel(
        _body,
        out_type=(
            jax.ShapeDtypeStruct((NC * NP, DH), jnp.float32),
            jax.ShapeDtypeStruct((NC * NP, DH), jnp.float32),
        ),
        mesh=mesh,
        compiler_params=pltpu.CompilerParams(use_tc_tiling_on_sc=False),
        scratch_types=[
            pltpu.VMEM((N_CHUNKS, CHUNK), jnp.int32),
            pltpu.VMEM((N_CHUNKS, CHUNK), jnp.int32),
            pltpu.VMEM((N_CHUNKS, CHUNK), jnp.float32),
            pltpu.VMEM((CHUNK, DH), jnp.float32),
            pltpu.VMEM((ZROWS, DH), jnp.float32),
            pltpu.VMEM_SHARED((NP, DH), jnp.float32),
        ],
    )(x2, col3, row3, vals3)

    return out.reshape(NC, NP, DH)[:, :N].transpose(1, 0, 2).reshape(N, D)
